# baseline (speedup 1.0000x reference)
import jax
import jax.numpy as jnp
from jax.experimental import pallas as pl
from jax.experimental.pallas import tpu as pltpu

B, C, H, W = 2, 768, 224, 224
S = H * W
SL, LN = 392, 128


def _index_kernel(yact_ref, idx_ref):
    yact = yact_ref[...]
    yact_t = jnp.transpose(yact)
    col_i = jax.lax.broadcasted_iota(jnp.int32, (C, C), 1)
    row_i = jax.lax.broadcasted_iota(jnp.int32, (C, C), 0)
    for b in range(B):
        vrow = jnp.broadcast_to(yact[b:b + 1, :], (C, C))
        vcol = jnp.broadcast_to(yact_t[:, b:b + 1], (C, C))
        before = jnp.logical_or(
            vrow > vcol,
            jnp.logical_and(vrow == vcol, col_i < row_i))
        rank_col = jnp.sum(before.astype(jnp.int32), axis=1, keepdims=True)
        hit = (jnp.broadcast_to(rank_col, (C, C)) == col_i)
        idx_row = jnp.sum(jnp.where(hit, row_i, 0), axis=0, keepdims=True)
        idx_ref[b:b + 1, :] = idx_row + b * C


def _compute_indices(yact):
    return pl.pallas_call(
        _index_kernel,
        in_specs=[pl.BlockSpec((B, C), lambda: (0, 0))],
        out_specs=pl.BlockSpec((B, C), lambda: (0, 0)),
        out_shape=jax.ShapeDtypeStruct((B, C), jnp.int32),
    )(yact)


NBUF = 8


def _gather_kernel(idx_ref, x_ref, out_ref, sems):
    g = pl.program_id(0)
    pltpu.make_async_copy(
        x_ref.at[idx_ref[g]], out_ref.at[g], sems.at[g % NBUF]).start()

    @pl.when(g >= NBUF - 1)
    def _drain_one():
        pltpu.make_async_copy(
            x_ref.at[0], out_ref.at[0], sems.at[(g + 1) % NBUF]).wait()

    @pl.when(g == B * C - 1)
    def _drain_rest():
        for k in range(NBUF - 1):
            pltpu.make_async_copy(
                x_ref.at[0], out_ref.at[0], sems.at[(g + 2 + k) % NBUF]).wait()


def _gather(x5, idx):
    grid_spec = pltpu.PrefetchScalarGridSpec(
        num_scalar_prefetch=1,
        grid=(B * C,),
        in_specs=[pl.BlockSpec(memory_space=pltpu.MemorySpace.HBM)],
        out_specs=pl.BlockSpec(memory_space=pltpu.MemorySpace.HBM),
        scratch_shapes=[pltpu.SemaphoreType.DMA((NBUF,))],
    )
    return pl.pallas_call(
        _gather_kernel,
        grid_spec=grid_spec,
        out_shape=jax.ShapeDtypeStruct((B * C, H, W), jnp.float32),
    )(idx.reshape(B * C), x5)


@jax.jit
def kernel(x, conv_w):
    x5 = x.reshape(B * C, H, W)
    y = jnp.mean(x, axis=(2, 3))
    yp = jnp.pad(y, ((0, 0), (1, 1)))
    wk = conv_w.reshape(3)
    yc = wk[0] * yp[:, :-2] + wk[1] * yp[:, 1:-1] + wk[2] * yp[:, 2:]
    yact = jax.nn.sigmoid(yc)
    idx = _compute_indices(yact)
    return _gather(x5, idx).reshape(B, C, H, W)


# speedup vs baseline: 34.2737x; 34.2737x over previous
import jax
import jax.numpy as jnp
from jax.experimental import pallas as pl
from jax.experimental.pallas import tpu as pltpu

B, C, H, W = 2, 768, 224, 224
S = H * W
RB = 1024
NJ = S // RB


def _index_kernel(yact_ref, idx_ref):
    yact = yact_ref[...]
    yact_t = jnp.transpose(yact)
    col_i = jax.lax.broadcasted_iota(jnp.int32, (C, C), 1)
    row_i = jax.lax.broadcasted_iota(jnp.int32, (C, C), 0)
    for b in range(B):
        vrow = jnp.broadcast_to(yact[b:b + 1, :], (C, C))
        vcol = jnp.broadcast_to(yact_t[:, b:b + 1], (C, C))
        before = jnp.logical_or(
            vrow > vcol,
            jnp.logical_and(vrow == vcol, col_i < row_i))
        rank_col = jnp.sum(before.astype(jnp.int32), axis=1, keepdims=True)
        hit = (jnp.broadcast_to(rank_col, (C, C)) == col_i)
        idx_row = jnp.sum(jnp.where(hit, row_i, 0), axis=0, keepdims=True)
        idx_ref[b:b + 1, :] = idx_row


def _compute_indices(yact):
    return pl.pallas_call(
        _index_kernel,
        in_specs=[pl.BlockSpec((B, C), lambda: (0, 0))],
        out_specs=pl.BlockSpec((B, C), lambda: (0, 0)),
        out_shape=jax.ShapeDtypeStruct((B, C), jnp.int32),
    )(yact)


def _permute_kernel(idx_ref, x_ref, o_ref):
    row_c = jax.lax.broadcasted_iota(jnp.int32, (C, C), 0)
    onehot = (jnp.broadcast_to(idx_ref[0], (C, C)) == row_c)
    perm = onehot.astype(jnp.float32)
    o_ref[...] = jnp.dot(x_ref[...], perm,
                         preferred_element_type=jnp.float32)


def _permute(xtf, idx):
    return pl.pallas_call(
        _permute_kernel,
        grid=(B, NJ),
        in_specs=[
            pl.BlockSpec((1, 1, C), lambda b, j: (b, 0, 0)),
            pl.BlockSpec((RB, C), lambda b, j: (b * NJ + j, 0)),
        ],
        out_specs=pl.BlockSpec((RB, C), lambda b, j: (b * NJ + j, 0)),
        out_shape=jax.ShapeDtypeStruct((B * S, C), jnp.float32),
    )(idx.reshape(B, 1, C), xtf)


@jax.jit
def kernel(x, conv_w):
    y = jnp.mean(x, axis=(2, 3))
    yp = jnp.pad(y, ((0, 0), (1, 1)))
    wk = conv_w.reshape(3)
    yc = wk[0] * yp[:, :-2] + wk[1] * yp[:, 1:-1] + wk[2] * yp[:, 2:]
    yact = jax.nn.sigmoid(yc)
    idx = _compute_indices(yact)
    xtf = jnp.transpose(x, (0, 2, 3, 1)).reshape(B * S, C)
    out = _permute(xtf, idx)
    return jnp.transpose(out.reshape(B, H, W, C), (0, 3, 1, 2))


# speedup vs baseline: 38.2970x; 1.1174x over previous
import jax
import jax.numpy as jnp
from jax.experimental import pallas as pl
from jax.experimental.pallas import tpu as pltpu

B, C, H, W = 2, 768, 224, 224
S = H * W
RB = 3136
NJ = S // RB


def _index_kernel(yact_ref, idx_ref):
    yact = yact_ref[...]
    yact_t = jnp.transpose(yact)
    col_i = jax.lax.broadcasted_iota(jnp.int32, (C, C), 1)
    row_i = jax.lax.broadcasted_iota(jnp.int32, (C, C), 0)
    for b in range(B):
        vrow = jnp.broadcast_to(yact[b:b + 1, :], (C, C))
        vcol = jnp.broadcast_to(yact_t[:, b:b + 1], (C, C))
        before = jnp.logical_or(
            vrow > vcol,
            jnp.logical_and(vrow == vcol, col_i < row_i))
        rank_col = jnp.sum(before.astype(jnp.int32), axis=1, keepdims=True)
        hit = (jnp.broadcast_to(rank_col, (C, C)) == col_i)
        idx_row = jnp.sum(jnp.where(hit, row_i, 0), axis=0, keepdims=True)
        idx_ref[b:b + 1, :] = idx_row


def _compute_indices(yact):
    return pl.pallas_call(
        _index_kernel,
        in_specs=[pl.BlockSpec((B, C), lambda: (0, 0))],
        out_specs=pl.BlockSpec((B, C), lambda: (0, 0)),
        out_shape=jax.ShapeDtypeStruct((B, C), jnp.int32),
    )(yact)


def _permute_kernel(idx_ref, x_ref, o_ref):
    row_c = jax.lax.broadcasted_iota(jnp.int32, (C, C), 0)
    onehot = (jnp.broadcast_to(idx_ref[0], (C, C)) == row_c)
    perm = onehot.astype(jnp.float32)
    o_ref[...] = jnp.dot(x_ref[...], perm,
                         preferred_element_type=jnp.float32)


def _permute(xtf, idx):
    return pl.pallas_call(
        _permute_kernel,
        grid=(B, NJ),
        in_specs=[
            pl.BlockSpec((1, 1, C), lambda b, j: (b, 0, 0)),
            pl.BlockSpec((RB, C), lambda b, j: (b * NJ + j, 0)),
        ],
        out_specs=pl.BlockSpec((RB, C), lambda b, j: (b * NJ + j, 0)),
        out_shape=jax.ShapeDtypeStruct((B * S, C), jnp.float32),
    )(idx.reshape(B, 1, C), xtf)


@jax.jit
def kernel(x, conv_w):
    y = jnp.mean(x, axis=(2, 3))
    yp = jnp.pad(y, ((0, 0), (1, 1)))
    wk = conv_w.reshape(3)
    yc = wk[0] * yp[:, :-2] + wk[1] * yp[:, 1:-1] + wk[2] * yp[:, 2:]
    yact = jax.nn.sigmoid(yc)
    idx = _compute_indices(yact)
    xtf = jnp.transpose(x, (0, 2, 3, 1)).reshape(B * S, C)
    out = _permute(xtf, idx)
    return jnp.transpose(out.reshape(B, H, W, C), (0, 3, 1, 2))


# speedup vs baseline: 38.7060x; 1.0107x over previous
import jax
import jax.numpy as jnp
from jax.experimental import pallas as pl
from jax.experimental.pallas import tpu as pltpu

B, C, H, W = 2, 768, 224, 224
S = H * W
RB = 3136
NJ = S // RB


def _permute_kernel(yact_ref, x_ref, o_ref, perm_ref):
    b = pl.program_id(0)
    j = pl.program_id(1)

    @pl.when(j == 0)
    def _build_perm():
        yact = yact_ref[...]
        yact_t = jnp.transpose(yact)
        col_i = jax.lax.broadcasted_iota(jnp.int32, (C, C), 1)
        row_i = jax.lax.broadcasted_iota(jnp.int32, (C, C), 0)
        is0 = (b == 0)
        vrow = jnp.broadcast_to(
            jnp.where(is0, yact[0:1, :], yact[1:2, :]), (C, C))
        vcol = jnp.broadcast_to(
            jnp.where(is0, yact_t[:, 0:1], yact_t[:, 1:2]), (C, C))
        before = jnp.logical_or(
            vrow > vcol,
            jnp.logical_and(vrow == vcol, col_i < row_i))
        rank_col = jnp.sum(before.astype(jnp.int32), axis=1, keepdims=True)
        perm_ref[...] = (
            jnp.broadcast_to(rank_col, (C, C)) == col_i).astype(jnp.float32)

    o_ref[...] = jnp.dot(x_ref[...], perm_ref[...],
                         preferred_element_type=jnp.float32)


def _permute(xtf, yact):
    return pl.pallas_call(
        _permute_kernel,
        grid=(B, NJ),
        in_specs=[
            pl.BlockSpec((B, C), lambda b, j: (0, 0)),
            pl.BlockSpec((RB, C), lambda b, j: (b * NJ + j, 0)),
        ],
        out_specs=pl.BlockSpec((RB, C), lambda b, j: (b * NJ + j, 0)),
        out_shape=jax.ShapeDtypeStruct((B * S, C), jnp.float32),
        scratch_shapes=[pltpu.VMEM((C, C), jnp.float32)],
    )(yact, xtf)


@jax.jit
def kernel(x, conv_w):
    y = jnp.mean(x, axis=(2, 3))
    yp = jnp.pad(y, ((0, 0), (1, 1)))
    wk = conv_w.reshape(3)
    yc = wk[0] * yp[:, :-2] + wk[1] * yp[:, 1:-1] + wk[2] * yp[:, 2:]
    yact = jax.nn.sigmoid(yc)
    xtf = jnp.transpose(x, (0, 2, 3, 1)).reshape(B * S, C)
    out = _permute(xtf, yact)
    return jnp.transpose(out.reshape(B, H, W, C), (0, 3, 1, 2))
